# baseline (speedup 1.0000x reference)
import functools

import jax
import jax.numpy as jnp
from jax import lax
from jax.experimental import pallas as pl
from jax.experimental.pallas import tpu as pltpu
from jax.experimental.pallas import tpu_sc as plsc

N_ROWS = 16384
ROW_W = 3200
D_IDX = 200
EMBED = 16
LANES = 128
SUBL = 8
N_TILE = ROW_W // LANES
EPT = LANES // EMBED

NC = 2
NS = 16
NW = NC * NS

N_CHUNK = 2
CH_ROWS = N_ROWS // N_CHUNK
N_G = CH_ROWS * D_IDX
G_PER_W = N_G // NW
BLK = SUBL * D_IDX
BLK_F = BLK * EMBED
N_BLK = G_PER_W // BLK
CHUNK = 128
N_FULL = BLK // CHUNK
TAIL = BLK - N_FULL * CHUNK
NBUF = 4
PREF = 2

BR_TC = 32
TC_GRID = CH_ROWS // BR_TC
TC_GRP = BR_TC // SUBL


def _sc_body(p_hbm, tab_hbm, pe_hbm, idx_v, pe_v, pe2_v, sem_ld, sem_st,
             sem_g):
    wid = lax.axis_index("s") * NC + lax.axis_index("c")
    g0 = wid * G_PER_W

    def start_load(blk_i, j):
        base = g0 + blk_i * BLK
        pltpu.async_copy(p_hbm.at[pl.ds(base, BLK)], idx_v.at[j],
                         sem_ld.at[j])

    def wait_load(blk_i, j):
        base = g0 + blk_i * BLK
        pltpu.make_async_copy(p_hbm.at[pl.ds(base, BLK)], idx_v.at[j],
                              sem_ld.at[j]).wait()

    def start_store(blk_i, j2):
        base = (g0 + blk_i * BLK) * EMBED
        pltpu.async_copy(pe2_v.at[j2], pe_hbm.at[pl.ds(base, BLK_F)],
                         sem_st.at[j2])

    def wait_store(blk_i, j2):
        base = (g0 + blk_i * BLK) * EMBED
        pltpu.make_async_copy(pe2_v.at[j2], pe_hbm.at[pl.ds(base, BLK_F)],
                              sem_st.at[j2]).wait()

    def gather_block(j):
        copies = []
        for c in range(N_FULL):
            copies.append(pltpu.async_copy(
                tab_hbm.at[idx_v.at[j, pl.ds(c * CHUNK, CHUNK)]],
                pe_v.at[pl.ds(c * CHUNK, CHUNK), :],
                sem_g,
            ))
        copies.append(pltpu.async_copy(
            tab_hbm.at[idx_v.at[j, pl.ds(N_FULL * CHUNK, TAIL)]],
            pe_v.at[pl.ds(N_FULL * CHUNK, TAIL), :],
            sem_g,
        ))
        for cp in copies:
            cp.wait()

    def permute_block(j2):
        def perm_t(t, carry):
            base2 = t * (SUBL * LANES)
            for r in range(SUBL):
                for u in range(EPT):
                    v = pe_v[r * D_IDX + t * EPT + u, :]
                    pe2_v[j2, pl.ds(pl.multiple_of(
                        base2 + r * LANES + u * EMBED, EMBED), EMBED)] = v
            return carry

        lax.fori_loop(0, N_TILE, perm_t, 0)

    for j in range(PREF):
        start_load(j, j)

    @pl.loop(0, N_BLK, step=NBUF)
    def blk_loop(b0):
        for j in range(NBUF):
            b = b0 + j
            j2 = j % 2
            nb = b + PREF

            @pl.when(nb < N_BLK)
            def _prefetch():
                start_load(nb, (j + PREF) % NBUF)

            wait_load(b, j)
            gather_block(j)

            @pl.when(b >= PREF)
            def _drain_store():
                wait_store(b - PREF, j2)

            permute_block(j2)
            start_store(b, j2)

    for b in range(N_BLK - PREF, N_BLK):
        wait_store(b, b % 2)


def _sc_gather(pf, tab):
    mesh = plsc.VectorSubcoreMesh(
        core_axis_name="c", subcore_axis_name="s", num_cores=NC,
        num_subcores=NS,
    )
    return pl.kernel(
        _sc_body,
        out_type=jax.ShapeDtypeStruct((N_G * EMBED,), jnp.float32),
        mesh=mesh,
        scratch_types=[
            pltpu.VMEM((NBUF, BLK), jnp.int32),
            pltpu.VMEM((BLK, EMBED), jnp.float32),
            pltpu.VMEM((2, BLK_F), jnp.float32),
            pltpu.SemaphoreType.DMA((NBUF,)),
            pltpu.SemaphoreType.DMA((2,)),
            pltpu.SemaphoreType.DMA,
        ],
        compiler_params=pltpu.CompilerParams(use_tc_tiling_on_sc=False),
    )(pf, tab)


def _tc_add_body(x_ref, pe_ref, o_ref):
    x = x_ref[...]
    pe = pe_ref[...].reshape(TC_GRP, N_TILE, SUBL, LANES)
    o_ref[...] = x + pe.transpose(0, 2, 1, 3).reshape(BR_TC, ROW_W)


def _tc_add(xk, pe1d):
    return pl.pallas_call(
        _tc_add_body,
        grid=(TC_GRID,),
        in_specs=[
            pl.BlockSpec((BR_TC, ROW_W), lambda i: (i, 0)),
            pl.BlockSpec((BR_TC * ROW_W,), lambda i: (i,)),
        ],
        out_specs=pl.BlockSpec((BR_TC, ROW_W), lambda i: (i, 0)),
        out_shape=jax.ShapeDtypeStruct((CH_ROWS, ROW_W), jnp.float32),
    )(xk, pe1d)


@functools.partial(jax.jit, static_argnames=())
def _run(x, p, tab):
    outs = []
    for k in range(N_CHUNK):
        pk = p[k * CH_ROWS:(k + 1) * CH_ROWS].reshape(N_G)
        pe1d = _sc_gather(pk, tab)
        outs.append(_tc_add(x[k * CH_ROWS:(k + 1) * CH_ROWS], pe1d))
    return jnp.concatenate(outs, axis=0)


def kernel(x, p, pos_table):
    return _run(x, p.astype(jnp.int32), pos_table)


# speedup vs baseline: 1.4790x; 1.4790x over previous
import functools

import jax
import jax.numpy as jnp
from jax import lax
from jax.experimental import pallas as pl
from jax.experimental.pallas import tpu as pltpu
from jax.experimental.pallas import tpu_sc as plsc

N_ROWS = 16384
ROW_W = 3200
D_IDX = 200
EMBED = 16

NC = 2
NS = 16
NW = NC * NS

N_G = N_ROWS * D_IDX
G_PER_W = N_G // NW
BLK = 1600
N_BLK = G_PER_W // BLK
CHUNK = 128
N_FULL = BLK // CHUNK
TAIL = BLK - N_FULL * CHUNK
NBUF = 4
PREF = 2

BR_TC = 32
TC_GRID = N_ROWS // BR_TC


def _sc_body(p_hbm, tab_hbm, pe_hbm, idx_v, pe_v, sem_ld, sem_st, sem_g):
    wid = lax.axis_index("s") * NC + lax.axis_index("c")
    g0 = wid * G_PER_W

    def start_load(blk_i, j):
        base = g0 + blk_i * BLK
        pltpu.async_copy(p_hbm.at[pl.ds(base, BLK)], idx_v.at[j],
                         sem_ld.at[j])

    def wait_load(blk_i, j):
        base = g0 + blk_i * BLK
        pltpu.make_async_copy(p_hbm.at[pl.ds(base, BLK)], idx_v.at[j],
                              sem_ld.at[j]).wait()

    def start_store(blk_i, j):
        base = g0 + blk_i * BLK
        pltpu.async_copy(pe_v.at[j], pe_hbm.at[pl.ds(base, BLK), :],
                         sem_st.at[j])

    def wait_store(blk_i, j):
        base = g0 + blk_i * BLK
        pltpu.make_async_copy(pe_v.at[j], pe_hbm.at[pl.ds(base, BLK), :],
                              sem_st.at[j]).wait()

    def gather_block(j):
        copies = []
        for c in range(N_FULL):
            copies.append(pltpu.async_copy(
                tab_hbm.at[idx_v.at[j, pl.ds(c * CHUNK, CHUNK)]],
                pe_v.at[j, pl.ds(c * CHUNK, CHUNK), :],
                sem_g,
            ))
        copies.append(pltpu.async_copy(
            tab_hbm.at[idx_v.at[j, pl.ds(N_FULL * CHUNK, TAIL)]],
            pe_v.at[j, pl.ds(N_FULL * CHUNK, TAIL), :],
            sem_g,
        ))
        for cp in copies:
            cp.wait()

    for j in range(PREF):
        start_load(j, j)

    @pl.loop(0, N_BLK, step=NBUF)
    def blk_loop(b0):
        for j in range(NBUF):
            b = b0 + j
            nb = b + PREF
            jn = (j + PREF) % NBUF

            @pl.when(nb < N_BLK)
            def _prefetch():
                @pl.when(b >= PREF)
                def _drain_store():
                    wait_store(b - PREF, jn)
                start_load(nb, jn)

            wait_load(b, j)
            gather_block(j)
            start_store(b, j)

    for b in range(N_BLK - NBUF, N_BLK):
        wait_store(b, b % NBUF)


def _sc_gather(pf, tab):
    mesh = plsc.VectorSubcoreMesh(
        core_axis_name="c", subcore_axis_name="s", num_cores=NC,
        num_subcores=NS,
    )
    return pl.kernel(
        _sc_body,
        out_type=jax.ShapeDtypeStruct((N_G, EMBED), jnp.float32),
        mesh=mesh,
        scratch_types=[
            pltpu.VMEM((NBUF, BLK), jnp.int32),
            pltpu.VMEM((NBUF, BLK, EMBED), jnp.float32),
            pltpu.SemaphoreType.DMA((NBUF,)),
            pltpu.SemaphoreType.DMA((NBUF,)),
            pltpu.SemaphoreType.DMA,
        ],
        compiler_params=pltpu.CompilerParams(use_tc_tiling_on_sc=False),
    )(pf, tab)


def _tc_add_body(x_ref, pe_ref, o_ref):
    for r in range(BR_TC):
        o_ref[pl.ds(r, 1), :] = (
            x_ref[pl.ds(r, 1), :]
            + pe_ref[pl.ds(r * ROW_W, ROW_W)].reshape(1, ROW_W)
        )


def _tc_add(x, pe1d):
    return pl.pallas_call(
        _tc_add_body,
        grid=(TC_GRID,),
        in_specs=[
            pl.BlockSpec((BR_TC, ROW_W), lambda i: (i, 0)),
            pl.BlockSpec((BR_TC * ROW_W,), lambda i: (i,)),
        ],
        out_specs=pl.BlockSpec((BR_TC, ROW_W), lambda i: (i, 0)),
        out_shape=jax.ShapeDtypeStruct((N_ROWS, ROW_W), jnp.float32),
    )(x, pe1d)


@functools.partial(jax.jit, static_argnames=())
def _run(x, p, tab):
    pe = _sc_gather(p.reshape(N_G), tab)
    return _tc_add(x, pe.reshape(N_G * EMBED))


def kernel(x, p, pos_table):
    return _run(x, p.astype(jnp.int32), pos_table)


# speedup vs baseline: 1.5962x; 1.0792x over previous
import functools

import jax
import jax.numpy as jnp
from jax import lax
from jax.experimental import pallas as pl
from jax.experimental.pallas import tpu as pltpu
from jax.experimental.pallas import tpu_sc as plsc

N_ROWS = 16384
ROW_W = 3200
D_IDX = 200
EMBED = 16

NC = 2
NS = 16
NW = NC * NS

N_CHUNK = 2
CH_ROWS = N_ROWS // N_CHUNK
N_G = CH_ROWS * D_IDX
G_PER_W = N_G // NW
BLK = 1600
N_BLK = G_PER_W // BLK
CHUNK = 128
N_FULL = BLK // CHUNK
TAIL = BLK - N_FULL * CHUNK
NBUF = 4
PREF = 2


def _sc_body(p_hbm, tab_hbm, pe_hbm, idx_v, pe_v, sem_ld, sem_st, sem_g):
    wid = lax.axis_index("s") * NC + lax.axis_index("c")
    g0 = wid * G_PER_W

    def start_load(blk_i, j):
        base = g0 + blk_i * BLK
        pltpu.async_copy(p_hbm.at[pl.ds(base, BLK)], idx_v.at[j],
                         sem_ld.at[j])

    def wait_load(blk_i, j):
        base = g0 + blk_i * BLK
        pltpu.make_async_copy(p_hbm.at[pl.ds(base, BLK)], idx_v.at[j],
                              sem_ld.at[j]).wait()

    def start_store(blk_i, j):
        base = g0 + blk_i * BLK
        pltpu.async_copy(pe_v.at[j], pe_hbm.at[pl.ds(base, BLK), :],
                         sem_st.at[j])

    def wait_store(blk_i, j):
        base = g0 + blk_i * BLK
        pltpu.make_async_copy(pe_v.at[j], pe_hbm.at[pl.ds(base, BLK), :],
                              sem_st.at[j]).wait()

    def gather_block(j):
        copies = []
        for c in range(N_FULL):
            copies.append(pltpu.async_copy(
                tab_hbm.at[idx_v.at[j, pl.ds(c * CHUNK, CHUNK)]],
                pe_v.at[j, pl.ds(c * CHUNK, CHUNK), :],
                sem_g,
            ))
        copies.append(pltpu.async_copy(
            tab_hbm.at[idx_v.at[j, pl.ds(N_FULL * CHUNK, TAIL)]],
            pe_v.at[j, pl.ds(N_FULL * CHUNK, TAIL), :],
            sem_g,
        ))
        for cp in copies:
            cp.wait()

    for j in range(PREF):
        start_load(j, j)

    @pl.loop(0, N_BLK, step=NBUF)
    def blk_loop(b0):
        for j in range(NBUF):
            b = b0 + j
            nb = b + PREF
            jn = (j + PREF) % NBUF

            @pl.when(nb < N_BLK)
            def _prefetch():
                @pl.when(b >= PREF)
                def _drain_store():
                    wait_store(b - PREF, jn)
                start_load(nb, jn)

            wait_load(b, j)
            gather_block(j)
            start_store(b, j)

    for b in range(N_BLK - NBUF, N_BLK):
        wait_store(b, b % NBUF)


def _sc_gather(pf, tab):
    mesh = plsc.VectorSubcoreMesh(
        core_axis_name="c", subcore_axis_name="s", num_cores=NC,
        num_subcores=NS,
    )
    return pl.kernel(
        _sc_body,
        out_type=jax.ShapeDtypeStruct((N_G, EMBED), jnp.float32),
        mesh=mesh,
        scratch_types=[
            pltpu.VMEM((NBUF, BLK), jnp.int32),
            pltpu.VMEM((NBUF, BLK, EMBED), jnp.float32),
            pltpu.SemaphoreType.DMA((NBUF,)),
            pltpu.SemaphoreType.DMA((NBUF,)),
            pltpu.SemaphoreType.DMA,
        ],
        compiler_params=pltpu.CompilerParams(use_tc_tiling_on_sc=False),
    )(pf, tab)


@functools.partial(jax.jit, static_argnames=())
def _run(x, p, tab):
    outs = []
    for k in range(N_CHUNK):
        pk = p[k * CH_ROWS:(k + 1) * CH_ROWS].reshape(N_G)
        pe = _sc_gather(pk, tab)
        xk = x[k * CH_ROWS:(k + 1) * CH_ROWS]
        outs.append(xk + pe.reshape(CH_ROWS, ROW_W))
    return jnp.concatenate(outs, axis=0)


def kernel(x, p, pos_table):
    return _run(x, p.astype(jnp.int32), pos_table)
